# baseline (speedup 1.0000x reference)
import functools

import jax
import jax.numpy as jnp
from jax.experimental import pallas as pl

S = 64.0
MEAN = 0.35
SIGMA = 0.0125

N_ROWS = 1024
N_COLS = 100000
BR = 1024


def _body(lab_ref, ela_ref, lt_ref, out_ref):
    r0 = pl.program_id(0) * BR
    rows = r0 + jax.lax.broadcasted_iota(jnp.int32, (BR, N_ROWS), 0)
    hit = rows == lab_ref[:, :]
    out_ref[:, :] = lt_ref[:, :] * S - jnp.where(hit, ela_ref[:, :], 0.0)


def kernel(logits, labels):
    ekey = jax.random.key(42)
    ela_s = (MEAN + SIGMA * jax.random.normal(ekey, (N_ROWS,), dtype=jnp.float32)) * S
    lt = jnp.swapaxes(logits, 0, 1)
    lab2 = labels.reshape(1, N_ROWS)
    ela2 = ela_s.reshape(1, N_ROWS)

    grid = (pl.cdiv(N_COLS, BR),)
    out_t = pl.pallas_call(
        _body,
        grid=grid,
        in_specs=[
            pl.BlockSpec((1, N_ROWS), lambda i: (0, 0)),
            pl.BlockSpec((1, N_ROWS), lambda i: (0, 0)),
            pl.BlockSpec((BR, N_ROWS), lambda i: (i, 0)),
        ],
        out_specs=pl.BlockSpec((BR, N_ROWS), lambda i: (i, 0)),
        out_shape=jax.ShapeDtypeStruct((N_COLS, N_ROWS), jnp.float32),
    )(lab2, ela2, lt)
    return jnp.swapaxes(out_t, 0, 1)


# speedup vs baseline: 1.0165x; 1.0165x over previous
import functools

import jax
import jax.numpy as jnp
from jax.experimental import pallas as pl

S = 64.0
MEAN = 0.35
SIGMA = 0.0125

N_ROWS = 1024
N_COLS = 100000
BR = 3072


def _body(lab_ref, ela_ref, lt_ref, out_ref):
    r0 = pl.program_id(0) * BR
    rows = r0 + jax.lax.broadcasted_iota(jnp.int32, (BR, N_ROWS), 0)
    hit = rows == lab_ref[:, :]
    out_ref[:, :] = lt_ref[:, :] * S - jnp.where(hit, ela_ref[:, :], 0.0)


def kernel(logits, labels):
    ekey = jax.random.key(42)
    ela_s = (MEAN + SIGMA * jax.random.normal(ekey, (N_ROWS,), dtype=jnp.float32)) * S
    lt = jnp.swapaxes(logits, 0, 1)
    lab2 = labels.reshape(1, N_ROWS)
    ela2 = ela_s.reshape(1, N_ROWS)

    grid = (pl.cdiv(N_COLS, BR),)
    out_t = pl.pallas_call(
        _body,
        grid=grid,
        in_specs=[
            pl.BlockSpec((1, N_ROWS), lambda i: (0, 0)),
            pl.BlockSpec((1, N_ROWS), lambda i: (0, 0)),
            pl.BlockSpec((BR, N_ROWS), lambda i: (i, 0)),
        ],
        out_specs=pl.BlockSpec((BR, N_ROWS), lambda i: (i, 0)),
        out_shape=jax.ShapeDtypeStruct((N_COLS, N_ROWS), jnp.float32),
    )(lab2, ela2, lt)
    return jnp.swapaxes(out_t, 0, 1)


# speedup vs baseline: 1.0168x; 1.0003x over previous
import functools

import jax
import jax.numpy as jnp
from jax.experimental import pallas as pl

S = 64.0
MEAN = 0.35
SIGMA = 0.0125

N_ROWS = 1024
N_COLS = 100000
BR = 3584


def _body(lab_ref, ela_ref, lt_ref, out_ref):
    r0 = pl.program_id(0) * BR
    rows = r0 + jax.lax.broadcasted_iota(jnp.int32, (BR, N_ROWS), 0)
    hit = rows == lab_ref[:, :]
    out_ref[:, :] = lt_ref[:, :] * S - jnp.where(hit, ela_ref[:, :], 0.0)


def kernel(logits, labels):
    ekey = jax.random.key(42)
    ela_s = (MEAN + SIGMA * jax.random.normal(ekey, (N_ROWS,), dtype=jnp.float32)) * S
    lt = jnp.swapaxes(logits, 0, 1)
    lab2 = labels.reshape(1, N_ROWS)
    ela2 = ela_s.reshape(1, N_ROWS)

    grid = (pl.cdiv(N_COLS, BR),)
    out_t = pl.pallas_call(
        _body,
        grid=grid,
        in_specs=[
            pl.BlockSpec((1, N_ROWS), lambda i: (0, 0)),
            pl.BlockSpec((1, N_ROWS), lambda i: (0, 0)),
            pl.BlockSpec((BR, N_ROWS), lambda i: (i, 0)),
        ],
        out_specs=pl.BlockSpec((BR, N_ROWS), lambda i: (i, 0)),
        out_shape=jax.ShapeDtypeStruct((N_COLS, N_ROWS), jnp.float32),
    )(lab2, ela2, lt)
    return jnp.swapaxes(out_t, 0, 1)


# speedup vs baseline: 1.0171x; 1.0002x over previous
import jax
import jax.numpy as jnp
from jax.experimental import pallas as pl

S = 64.0
MEAN = 0.35
SIGMA = 0.0125

N_ROWS = 1024
N_COLS = 100000
BR = 3072


def _body(lab_ref, ela_ref, lt_ref, out_ref):
    r0 = pl.program_id(0) * BR
    rows = r0 + jax.lax.broadcasted_iota(jnp.int32, (BR, N_ROWS), 0)
    hit = rows == lab_ref[:, :]
    out_ref[:, :] = lt_ref[:, :] * S - jnp.where(hit, ela_ref[:, :], 0.0)


def kernel(logits, labels):
    ekey = jax.random.key(42)
    ela_s = (MEAN + SIGMA * jax.random.normal(ekey, (N_ROWS,), dtype=jnp.float32)) * S
    lt = jnp.swapaxes(logits, 0, 1)
    lab2 = labels.reshape(1, N_ROWS)
    ela2 = ela_s.reshape(1, N_ROWS)

    grid = (pl.cdiv(N_COLS, BR),)
    out_t = pl.pallas_call(
        _body,
        grid=grid,
        in_specs=[
            pl.BlockSpec((1, N_ROWS), lambda i: (0, 0)),
            pl.BlockSpec((1, N_ROWS), lambda i: (0, 0)),
            pl.BlockSpec((BR, N_ROWS), lambda i: (i, 0)),
        ],
        out_specs=pl.BlockSpec((BR, N_ROWS), lambda i: (i, 0)),
        out_shape=jax.ShapeDtypeStruct((N_COLS, N_ROWS), jnp.float32),
    )(lab2, ela2, lt)
    return jnp.swapaxes(out_t, 0, 1)
